# baseline (speedup 1.0000x reference)
import jax
import jax.numpy as jnp
from jax import lax
from jax.experimental import pallas as pl
from jax.experimental.pallas import tpu as pltpu
from jax.experimental.pallas import tpu_sc as plsc

N_NODES = 100000
D_FEAT = 256
N_GRAPHS = 512
N_CORES = 2
N_SUBCORES = 16
COLS = D_FEAT // N_CORES
ROWS_PER_T = N_NODES // N_SUBCORES
LANES = 16
KVECS = COLS // LANES
N_GROUPS = (ROWS_PER_T + LANES - 1) // LANES
CHUNK = 16
NBUF = 12
TRASH = N_GRAPHS
ACC_ROWS = 513
OUT_STRIPE = N_GRAPHS // N_SUBCORES
STAGE = ROWS_PER_T + 6
IDXBUF = STAGE + 10
CBUF = ROWS_PER_T + CHUNK + 22


def _sc_body(x_hbm, type_hbm, batch_hbm, out_hbm, part_hbm,
             type_v, batch_v, crow, cseg, xbuf, accum, tmp, rbuf,
             gsem0, gsem1, gsem2, gsem3, gsem4, gsem5, gsem6, gsem7,
             gsem8, gsem9, gsem10, gsem11, rsem0, rsem1):
    c = lax.axis_index("c")
    s = lax.axis_index("s")
    rbase = s * ROWS_PER_T
    shift = lax.rem(rbase, 8)
    abase = pl.multiple_of(rbase - shift, 8)
    cbase = pl.multiple_of(c * COLS, COLS)

    zero16 = jnp.zeros((LANES,), jnp.float32)
    iota16 = lax.iota(jnp.int32, LANES)

    pltpu.async_copy(type_hbm.at[pl.ds(abase, STAGE)],
                     type_v.at[pl.ds(0, STAGE)], rsem0)
    pltpu.async_copy(batch_hbm.at[pl.ds(abase, STAGE)],
                     batch_v.at[pl.ds(0, STAGE)], rsem1)

    def zacc(r, carry):
        for k in range(KVECS):
            accum[r, pl.ds(k * LANES, LANES)] = zero16
        return carry

    lax.fori_loop(0, ACC_ROWS, zacc, 0)
    pltpu.make_async_copy(type_hbm.at[pl.ds(abase, STAGE)],
                          type_v.at[pl.ds(0, STAGE)], rsem0).wait()
    pltpu.make_async_copy(batch_hbm.at[pl.ds(abase, STAGE)],
                          batch_v.at[pl.ds(0, STAGE)], rsem1).wait()

    def gather(coff, buf, sem):
        pltpu.async_copy(
            x_hbm.at[crow.at[pl.ds(coff, CHUNK)], pl.ds(cbase, COLS)],
            buf, sem)

    gsems = (gsem0, gsem1, gsem2, gsem3, gsem4, gsem5, gsem6, gsem7,
             gsem8, gsem9, gsem10, gsem11)

    def cgroup(gi, carry):
        cnt, fired = carry
        o = gi * LANES + shift
        t16 = type_v[pl.ds(o, LANES)]
        seg16 = batch_v[pl.ds(o, LANES)]
        nvalid = jnp.minimum(ROWS_PER_T - gi * LANES, LANES)
        mask = jnp.logical_and(t16 == 0, iota16 < nvalid)
        rid16 = rbase + gi * LANES + iota16
        mi = mask.astype(jnp.int32)
        incl = jnp.cumsum(mi)
        pos = jnp.where(mask, cnt + incl - mi, CBUF - 1)
        plsc.store_scatter(crow, [pos], rid16)
        plsc.store_scatter(cseg, [pos], seg16)
        cnt = cnt + incl[LANES - 1]
        can_fire = jnp.logical_and(fired < NBUF - 1,
                                   (fired + 1) * CHUNK <= cnt)

        for b in range(NBUF - 1):
            @pl.when(jnp.logical_and(can_fire, fired == b))
            def _(b=b):
                gather(pl.multiple_of(jnp.int32(b * CHUNK), 8),
                       xbuf.at[b], gsems[b])

        return cnt, fired + can_fire.astype(jnp.int32)

    cnt, fired = lax.fori_loop(0, N_GROUPS, cgroup,
                               (jnp.int32(0), jnp.int32(0)))

    for k in range(CHUNK // LANES):
        crow[pl.ds(cnt + k * LANES, LANES)] = jnp.zeros((LANES,), jnp.int32)
        cseg[pl.ds(cnt + k * LANES, LANES)] = jnp.full((LANES,), TRASH,
                                                       jnp.int32)

    n_chunks = (cnt + CHUNK - 1) // CHUNK

    def gwait(buf, sem):
        pltpu.make_async_copy(
            x_hbm.at[crow.at[pl.ds(0, CHUNK)], pl.ds(cbase, COLS)],
            buf, sem).wait()

    for b in range(NBUF - 1):
        @pl.when(jnp.logical_and(jnp.int32(b) >= fired,
                                 jnp.int32(b) < n_chunks))
        def _(b=b):
            gather(pl.multiple_of(jnp.int32(b * CHUNK), 8),
                   xbuf.at[b], gsems[b])

    def chunk_body(ci, carry):
        par = lax.rem(ci, NBUF)
        noff = pl.multiple_of((ci + NBUF - 1) * CHUNK, 8)

        for b in range(NBUF):
            @pl.when(par == b)
            def _(b=b):
                gwait(xbuf.at[b], gsems[b])
                nb = (b + NBUF - 1) % NBUF

                @pl.when(ci + NBUF - 1 < n_chunks)
                def _(b=b, nb=nb):
                    gather(noff, xbuf.at[nb], gsems[nb])

        coff = pl.multiple_of(ci * CHUNK, 8)

        def group_body(gi, carry):
            prev = carry[0]
            seg16 = cseg[pl.ds(coff + gi * LANES, LANES)]
            one_run = jnp.all(seg16 == prev)

            def fast(carry):
                prev, *acc = carry
                for r2 in range(LANES):
                    acc = [acc[k] + xbuf[par, gi * LANES + r2,
                                         pl.ds(k * LANES, LANES)]
                           for k in range(KVECS)]
                return (prev, *acc)

            def slow(carry):
                prev, *acc = carry
                for r2 in range(LANES):
                    seg = seg16[r2]
                    same = seg == prev
                    new_acc = []
                    for k in range(KVECS):
                        accum[prev, pl.ds(k * LANES, LANES)] = acc[k]
                        a = jnp.where(same, acc[k], zero16)
                        a = a + xbuf[par, gi * LANES + r2,
                                     pl.ds(k * LANES, LANES)]
                        accum[seg, pl.ds(k * LANES, LANES)] = a
                        new_acc.append(a)
                    acc = new_acc
                    prev = seg
                return (prev, *acc)

            return lax.cond(one_run, fast, slow, carry)

        return lax.fori_loop(0, CHUNK // LANES, group_body, carry)

    carry0 = (jnp.int32(TRASH),) + (zero16,) * KVECS
    fprev, *facc = lax.fori_loop(0, n_chunks, chunk_body, carry0)
    for k in range(KVECS):
        accum[fprev, pl.ds(k * LANES, LANES)] = facc[k]

    pltpu.sync_copy(accum.at[pl.ds(0, N_GRAPHS)], part_hbm.at[c].at[s])
    plsc.subcore_barrier()

    def stripe(t):
        return part_hbm.at[c].at[t].at[pl.ds(s * OUT_STRIPE, OUT_STRIPE)]

    pltpu.async_copy(stripe(jnp.int32(0)), tmp.at[0], rsem0)

    def tsum(t, carry):
        par = lax.rem(t, 2)

        @pl.when(par == 0)
        def _():
            pltpu.make_async_copy(stripe(t), tmp.at[0], rsem0).wait()

            @pl.when(t + 1 < N_SUBCORES)
            def _():
                pltpu.async_copy(stripe(t + 1), tmp.at[1], rsem1)

        @pl.when(par == 1)
        def _():
            pltpu.make_async_copy(stripe(t), tmp.at[1], rsem1).wait()

            @pl.when(t + 1 < N_SUBCORES)
            def _():
                pltpu.async_copy(stripe(t + 1), tmp.at[0], rsem0)

        def radd(r, carry2):
            for k in range(KVECS):
                sl = pl.ds(k * LANES, LANES)
                v = tmp[par, r, sl]
                rbuf[r, sl] = jnp.where(t == 0, v, rbuf[r, sl] + v)
            return carry2

        lax.fori_loop(0, OUT_STRIPE, radd, 0)
        return carry

    lax.fori_loop(0, N_SUBCORES, tsum, 0)

    pltpu.sync_copy(rbuf, out_hbm.at[pl.ds(s * OUT_STRIPE, OUT_STRIPE),
                                     pl.ds(cbase, COLS)])


_mesh = plsc.VectorSubcoreMesh(core_axis_name="c", subcore_axis_name="s",
                               num_cores=N_CORES, num_subcores=N_SUBCORES)

_sc_call = pl.kernel(
    _sc_body,
    out_type=(jax.ShapeDtypeStruct((N_GRAPHS, D_FEAT), jnp.float32),
              jax.ShapeDtypeStruct((N_CORES, N_SUBCORES, N_GRAPHS, COLS),
                                   jnp.float32)),
    mesh=_mesh,
    compiler_params=pltpu.CompilerParams(needs_layout_passes=False),
    scratch_types=[
        pltpu.VMEM((IDXBUF,), jnp.int32),
        pltpu.VMEM((IDXBUF,), jnp.int32),
        pltpu.VMEM((CBUF,), jnp.int32),
        pltpu.VMEM((CBUF,), jnp.int32),
        pltpu.VMEM((NBUF, CHUNK, COLS), jnp.float32),
        pltpu.VMEM((ACC_ROWS, COLS), jnp.float32),
        pltpu.VMEM((2, OUT_STRIPE, COLS), jnp.float32),
        pltpu.VMEM((OUT_STRIPE, COLS), jnp.float32),
        pltpu.SemaphoreType.DMA,
        pltpu.SemaphoreType.DMA,
        pltpu.SemaphoreType.DMA,
        pltpu.SemaphoreType.DMA,
        pltpu.SemaphoreType.DMA,
        pltpu.SemaphoreType.DMA,
        pltpu.SemaphoreType.DMA,
        pltpu.SemaphoreType.DMA,
        pltpu.SemaphoreType.DMA,
        pltpu.SemaphoreType.DMA,
        pltpu.SemaphoreType.DMA,
        pltpu.SemaphoreType.DMA,
        pltpu.SemaphoreType.DMA,
        pltpu.SemaphoreType.DMA,
    ],
)


@jax.jit
def kernel(x, atom_origin_type, batch):
    t = atom_origin_type.astype(jnp.int32)
    b = batch.astype(jnp.int32)
    out, _ = _sc_call(x, t, b)
    return out


# speedup vs baseline: 1.2242x; 1.2242x over previous
import jax
import jax.numpy as jnp
from jax import lax
from jax.experimental import pallas as pl
from jax.experimental.pallas import tpu as pltpu
from jax.experimental.pallas import tpu_sc as plsc

N_NODES = 100000
D_FEAT = 256
N_GRAPHS = 512
N_CORES = 2
N_SUBCORES = 16
COLS = D_FEAT // N_CORES
ROWS_PER_T = N_NODES // N_SUBCORES
LANES = 16
KVECS = COLS // LANES
N_GROUPS = (ROWS_PER_T + LANES - 1) // LANES
CHUNK = 16
NBUF = 12
TRASH = N_GRAPHS
ACC_ROWS = 513
OUT_STRIPE = N_GRAPHS // N_SUBCORES
STAGE = ROWS_PER_T + 6
IDXBUF = STAGE + 10
CBUF = ROWS_PER_T + CHUNK + 22


def _sc_body(x_hbm, type_hbm, batch_hbm, out_hbm, part_hbm,
             type_v, batch_v, crow, cseg, xbuf, accum, tmp, rbuf,
             gsem0, gsem1, gsem2, gsem3, gsem4, gsem5, gsem6, gsem7,
             gsem8, gsem9, gsem10, gsem11, rsem0, rsem1):
    c = lax.axis_index("c")
    s = lax.axis_index("s")
    rbase = s * ROWS_PER_T
    shift = lax.rem(rbase, 8)
    abase = pl.multiple_of(rbase - shift, 8)
    cbase = pl.multiple_of(c * COLS, COLS)

    zero16 = jnp.zeros((LANES,), jnp.float32)
    iota16 = lax.iota(jnp.int32, LANES)

    pltpu.async_copy(type_hbm.at[pl.ds(abase, STAGE)],
                     type_v.at[pl.ds(0, STAGE)], rsem0)
    pltpu.async_copy(batch_hbm.at[pl.ds(abase, STAGE)],
                     batch_v.at[pl.ds(0, STAGE)], rsem1)

    def zacc(r, carry):
        for k in range(KVECS):
            accum[r, pl.ds(k * LANES, LANES)] = zero16
        return carry

    lax.fori_loop(0, ACC_ROWS, zacc, 0)
    pltpu.make_async_copy(type_hbm.at[pl.ds(abase, STAGE)],
                          type_v.at[pl.ds(0, STAGE)], rsem0).wait()
    pltpu.make_async_copy(batch_hbm.at[pl.ds(abase, STAGE)],
                          batch_v.at[pl.ds(0, STAGE)], rsem1).wait()

    def cgroup(gi, cnt):
        o = gi * LANES + shift
        t16 = type_v[pl.ds(o, LANES)]
        seg16 = batch_v[pl.ds(o, LANES)]
        nvalid = jnp.minimum(ROWS_PER_T - gi * LANES, LANES)
        mask = jnp.logical_and(t16 == 0, iota16 < nvalid)
        rid16 = rbase + gi * LANES + iota16
        mi = mask.astype(jnp.int32)
        incl = jnp.cumsum(mi)
        pos = jnp.where(mask, cnt + incl - mi, CBUF - 1)
        plsc.store_scatter(crow, [pos], rid16)
        plsc.store_scatter(cseg, [pos], seg16)
        return cnt + incl[LANES - 1]

    cnt = lax.fori_loop(0, N_GROUPS, cgroup, jnp.int32(0))

    for k in range(CHUNK // LANES):
        crow[pl.ds(cnt + k * LANES, LANES)] = jnp.zeros((LANES,), jnp.int32)
        cseg[pl.ds(cnt + k * LANES, LANES)] = jnp.full((LANES,), TRASH,
                                                       jnp.int32)

    n_chunks = (cnt + CHUNK - 1) // CHUNK

    def gather(coff, buf, sem):
        pltpu.async_copy(
            x_hbm.at[crow.at[pl.ds(coff, CHUNK)], pl.ds(cbase, COLS)],
            buf, sem)

    def gwait(buf, sem):
        pltpu.make_async_copy(
            x_hbm.at[crow.at[pl.ds(0, CHUNK)], pl.ds(cbase, COLS)],
            buf, sem).wait()

    gsems = (gsem0, gsem1, gsem2, gsem3, gsem4, gsem5, gsem6, gsem7,
             gsem8, gsem9, gsem10, gsem11)
    for b in range(NBUF - 1):
        @pl.when(jnp.int32(b) < n_chunks)
        def _(b=b):
            gather(pl.multiple_of(jnp.int32(b * CHUNK), 8),
                   xbuf.at[b], gsems[b])

    def chunk_body(ci, carry):
        par = lax.rem(ci, NBUF)
        noff = pl.multiple_of((ci + NBUF - 1) * CHUNK, 8)

        for b in range(NBUF):
            @pl.when(par == b)
            def _(b=b):
                gwait(xbuf.at[b], gsems[b])
                nb = (b + NBUF - 1) % NBUF

                @pl.when(ci + NBUF - 1 < n_chunks)
                def _(b=b, nb=nb):
                    gather(noff, xbuf.at[nb], gsems[nb])

        coff = pl.multiple_of(ci * CHUNK, 8)

        def group_body(gi, carry):
            prev = carry[0]
            seg16 = cseg[pl.ds(coff + gi * LANES, LANES)]
            one_run = jnp.all(seg16 == prev)

            def fast(carry):
                prev, *acc = carry
                for r2 in range(LANES):
                    acc = [acc[k] + xbuf[par, gi * LANES + r2,
                                         pl.ds(k * LANES, LANES)]
                           for k in range(KVECS)]
                return (prev, *acc)

            def slow(carry):
                prev, *acc = carry
                for r2 in range(LANES):
                    seg = seg16[r2]
                    same = seg == prev
                    new_acc = []
                    for k in range(KVECS):
                        accum[prev, pl.ds(k * LANES, LANES)] = acc[k]
                        a = jnp.where(same, acc[k], zero16)
                        a = a + xbuf[par, gi * LANES + r2,
                                     pl.ds(k * LANES, LANES)]
                        accum[seg, pl.ds(k * LANES, LANES)] = a
                        new_acc.append(a)
                    acc = new_acc
                    prev = seg
                return (prev, *acc)

            return lax.cond(one_run, fast, slow, carry)

        return lax.fori_loop(0, CHUNK // LANES, group_body, carry)

    carry0 = (jnp.int32(TRASH),) + (zero16,) * KVECS
    fprev, *facc = lax.fori_loop(0, n_chunks, chunk_body, carry0)
    for k in range(KVECS):
        accum[fprev, pl.ds(k * LANES, LANES)] = facc[k]

    pltpu.sync_copy(accum.at[pl.ds(0, N_GRAPHS)], part_hbm.at[c].at[s])
    plsc.subcore_barrier()

    def stripe(t):
        return part_hbm.at[c].at[t].at[pl.ds(s * OUT_STRIPE, OUT_STRIPE)]

    pltpu.async_copy(stripe(jnp.int32(0)), tmp.at[0], rsem0)

    def tsum(t, carry):
        par = lax.rem(t, 2)

        @pl.when(par == 0)
        def _():
            pltpu.make_async_copy(stripe(t), tmp.at[0], rsem0).wait()

            @pl.when(t + 1 < N_SUBCORES)
            def _():
                pltpu.async_copy(stripe(t + 1), tmp.at[1], rsem1)

        @pl.when(par == 1)
        def _():
            pltpu.make_async_copy(stripe(t), tmp.at[1], rsem1).wait()

            @pl.when(t + 1 < N_SUBCORES)
            def _():
                pltpu.async_copy(stripe(t + 1), tmp.at[0], rsem0)

        def radd(r, carry2):
            for k in range(KVECS):
                sl = pl.ds(k * LANES, LANES)
                v = tmp[par, r, sl]
                rbuf[r, sl] = jnp.where(t == 0, v, rbuf[r, sl] + v)
            return carry2

        lax.fori_loop(0, OUT_STRIPE, radd, 0)
        return carry

    lax.fori_loop(0, N_SUBCORES, tsum, 0)

    pltpu.sync_copy(rbuf, out_hbm.at[pl.ds(s * OUT_STRIPE, OUT_STRIPE),
                                     pl.ds(cbase, COLS)])


_mesh = plsc.VectorSubcoreMesh(core_axis_name="c", subcore_axis_name="s",
                               num_cores=N_CORES, num_subcores=N_SUBCORES)

_sc_call = pl.kernel(
    _sc_body,
    out_type=(jax.ShapeDtypeStruct((N_GRAPHS, D_FEAT), jnp.float32),
              jax.ShapeDtypeStruct((N_CORES, N_SUBCORES, N_GRAPHS, COLS),
                                   jnp.float32)),
    mesh=_mesh,
    compiler_params=pltpu.CompilerParams(needs_layout_passes=False),
    scratch_types=[
        pltpu.VMEM((IDXBUF,), jnp.int32),
        pltpu.VMEM((IDXBUF,), jnp.int32),
        pltpu.VMEM((CBUF,), jnp.int32),
        pltpu.VMEM((CBUF,), jnp.int32),
        pltpu.VMEM((NBUF, CHUNK, COLS), jnp.float32),
        pltpu.VMEM((ACC_ROWS, COLS), jnp.float32),
        pltpu.VMEM((2, OUT_STRIPE, COLS), jnp.float32),
        pltpu.VMEM((OUT_STRIPE, COLS), jnp.float32),
        pltpu.SemaphoreType.DMA,
        pltpu.SemaphoreType.DMA,
        pltpu.SemaphoreType.DMA,
        pltpu.SemaphoreType.DMA,
        pltpu.SemaphoreType.DMA,
        pltpu.SemaphoreType.DMA,
        pltpu.SemaphoreType.DMA,
        pltpu.SemaphoreType.DMA,
        pltpu.SemaphoreType.DMA,
        pltpu.SemaphoreType.DMA,
        pltpu.SemaphoreType.DMA,
        pltpu.SemaphoreType.DMA,
        pltpu.SemaphoreType.DMA,
        pltpu.SemaphoreType.DMA,
    ],
)


@jax.jit
def kernel(x, atom_origin_type, batch):
    t = atom_origin_type.astype(jnp.int32)
    b = batch.astype(jnp.int32)
    out, _ = _sc_call(x, t, b)
    return out
